# baseline (speedup 1.0000x reference)
import jax
import jax.numpy as jnp
from jax import lax
from jax.experimental import pallas as pl
from jax.experimental.pallas import tpu as pltpu
from jax.experimental.pallas import tpu_sc as plsc

_N = 4096 * 200
_NW = 32
_PER_W = _N // _NW
_C = 1024
_LANES = 16
_COLS = 24


def _sc_body(d_hbm, g_hbm, out_hbm, g_v, d_v, o_v):
    wid = lax.axis_index("s") * 2 + lax.axis_index("c")
    pltpu.sync_copy(g_hbm, g_v)
    base_w = wid * _PER_W
    iota = lax.broadcasted_iota(jnp.int32, (_LANES,), 0)

    def chunk_body(ci, carry):
        base = base_w + ci * _C
        pltpu.sync_copy(d_hbm.at[pl.ds(base, _C)], d_v)

        def grp(g, c2):
            dv = d_v[pl.ds(g * _LANES, _LANES)]
            idxf = jnp.minimum(jnp.maximum(dv, 0.0), 1.0 - 1e-6) * 255.0
            i0 = idxf.astype(jnp.int32)
            al = idxf - i0.astype(jnp.float32)
            pos0 = g * (_LANES * _COLS) + iota * _COLS
            row0 = i0 * (2 * _COLS)
            for j in range(_COLS):
                v0 = plsc.load_gather(g_v, [row0 + j])
                v1 = plsc.load_gather(g_v, [row0 + (j + _COLS)])
                plsc.store_scatter(o_v, [pos0 + j], v0 + al * v1)
            return c2

        lax.fori_loop(0, _C // _LANES, grp, 0)
        pltpu.sync_copy(o_v, out_hbm.at[pl.ds(base * _COLS, _C * _COLS)])
        return carry

    lax.fori_loop(0, _PER_W // _C, chunk_body, 0)


def kernel(distances, basis_values):
    num_basis, num_functions, domain_size = basis_values.shape
    orig_shape = distances.shape
    n = distances.size
    t = basis_values.transpose(2, 0, 1).reshape(domain_size, _COLS)
    delta = jnp.concatenate(
        [t[1:] - t[:-1], jnp.zeros((1, _COLS), jnp.float32)], axis=0)
    g = jnp.concatenate([t, delta], axis=1).reshape(-1)

    mesh = plsc.VectorSubcoreMesh(core_axis_name="c", subcore_axis_name="s")
    out = pl.kernel(
        _sc_body,
        out_type=jax.ShapeDtypeStruct((n * _COLS,), jnp.float32),
        mesh=mesh,
        compiler_params=pltpu.CompilerParams(needs_layout_passes=False),
        scratch_types=[
            pltpu.VMEM((domain_size * 2 * _COLS,), jnp.float32),
            pltpu.VMEM((_C,), jnp.float32),
            pltpu.VMEM((_C * _COLS,), jnp.float32),
        ],
    )(distances.reshape(n), g)
    return out.reshape(*orig_shape, num_basis, num_functions)


# speedup vs baseline: 1.0146x; 1.0146x over previous
import jax
import jax.numpy as jnp
from jax import lax
from jax.experimental import pallas as pl
from jax.experimental.pallas import tpu as pltpu
from jax.experimental.pallas import tpu_sc as plsc

_N = 4096 * 200
_NW = 32
_PER_W = _N // _NW
_C = 1024
_LANES = 16
_COLS = 24


def _sc_body(d_hbm, g_hbm, out_hbm, g_v, d_v, o_v):
    wid = lax.axis_index("s") * 2 + lax.axis_index("c")
    pltpu.sync_copy(g_hbm, g_v)
    base_w = wid * _PER_W
    iota = lax.broadcasted_iota(jnp.int32, (_LANES,), 0)

    def chunk_body(ci, carry):
        base = base_w + ci * _C
        pltpu.sync_copy(d_hbm.at[pl.ds(base, _C)], d_v)

        @plsc.parallel_loop(0, _C // _LANES, unroll=2)
        def grp(g):
            dv = d_v[pl.ds(g * _LANES, _LANES)]
            idxf = jnp.minimum(jnp.maximum(dv, 0.0), 1.0 - 1e-6) * 255.0
            i0 = idxf.astype(jnp.int32)
            al = idxf - i0.astype(jnp.float32)
            pos0 = g * (_LANES * _COLS) + iota * _COLS
            row0 = i0 * (2 * _COLS)
            for j in range(_COLS):
                v0 = plsc.load_gather(g_v, [row0 + j])
                v1 = plsc.load_gather(g_v, [row0 + (j + _COLS)])
                plsc.store_scatter(o_v, [pos0 + j], v0 + al * v1)
        pltpu.sync_copy(o_v, out_hbm.at[pl.ds(base * _COLS, _C * _COLS)])
        return carry

    lax.fori_loop(0, _PER_W // _C, chunk_body, 0)


def kernel(distances, basis_values):
    num_basis, num_functions, domain_size = basis_values.shape
    orig_shape = distances.shape
    n = distances.size
    t = basis_values.transpose(2, 0, 1).reshape(domain_size, _COLS)
    delta = jnp.concatenate(
        [t[1:] - t[:-1], jnp.zeros((1, _COLS), jnp.float32)], axis=0)
    g = jnp.concatenate([t, delta], axis=1).reshape(-1)

    mesh = plsc.VectorSubcoreMesh(core_axis_name="c", subcore_axis_name="s")
    out = pl.kernel(
        _sc_body,
        out_type=jax.ShapeDtypeStruct((n * _COLS,), jnp.float32),
        mesh=mesh,
        compiler_params=pltpu.CompilerParams(needs_layout_passes=False),
        scratch_types=[
            pltpu.VMEM((domain_size * 2 * _COLS,), jnp.float32),
            pltpu.VMEM((_C,), jnp.float32),
            pltpu.VMEM((_C * _COLS,), jnp.float32),
        ],
    )(distances.reshape(n), g)
    return out.reshape(*orig_shape, num_basis, num_functions)
